# Initial kernel scaffold; baseline (speedup 1.0000x reference)
import jax
import jax.numpy as jnp
from jax import lax
from jax.experimental import pallas as pl
from jax.experimental.pallas import tpu as pltpu

_NUM_CLASSES = 3
_POS_T, _NEG_T = 0.5, 0.3
_RATIO = 3
_A = 3
_G = 32
_HW = 128
_KD = 5 + _NUM_CLASSES


def _smooth_l1(x, y):
    d = jnp.abs(x - y)
    return jnp.where(d < 1.0, 0.5 * d * d, d - 0.5)


def _loss_kernel(pred_ref, anc_ref, gtb_ref, lab_ref, out_ref):
    b = pl.program_id(0)
    f32 = jnp.float32

    num_pos = f32(0.0)
    num_neg = f32(0.0)
    obj_pos_sum = f32(0.0)
    cls_sum = f32(0.0)
    loc_sum = f32(0.0)

    neg_bits = []
    obj_losses = []

    for a in range(_A):
        ax1 = anc_ref[a, 0]
        ay1 = anc_ref[a, 1]
        ax2 = anc_ref[a, 2]
        ay2 = anc_ref[a, 3]
        area_a = jnp.maximum(ax2 - ax1, 0.0) * jnp.maximum(ay2 - ay1, 0.0)
        aw = jnp.maximum(ax2 - ax1, 1e-6)
        ah = jnp.maximum(ay2 - ay1, 1e-6)
        axc = (ax1 + ax2) * 0.5
        ayc = (ay1 + ay2) * 0.5

        def iou_body(g, carry, ax1=ax1, ay1=ay1, ax2=ax2, ay2=ay2,
                     area_a=area_a):
            best, bg = carry
            base = 4 * g
            bx1 = gtb_ref[0, base]
            by1 = gtb_ref[0, base + 1]
            bx2 = gtb_ref[0, base + 2]
            by2 = gtb_ref[0, base + 3]
            ix1 = jnp.maximum(ax1, bx1)
            iy1 = jnp.maximum(ay1, by1)
            ix2 = jnp.minimum(ax2, bx2)
            iy2 = jnp.minimum(ay2, by2)
            inter = jnp.maximum(ix2 - ix1, 0.0) * jnp.maximum(iy2 - iy1, 0.0)
            ab = jnp.maximum(bx2 - bx1, 0.0) * jnp.maximum(by2 - by1, 0.0)
            union = area_a + ab - inter
            iou = inter / jnp.maximum(union, 1e-9)
            upd = iou > best
            best = jnp.where(upd, iou, best)
            bg = jnp.where(upd, g.astype(f32), bg)
            return best, bg

        init = (jnp.full((_HW, _HW), -1.0, f32), jnp.zeros((_HW, _HW), f32))
        best, bg = lax.fori_loop(0, _G, iou_body, init)

        posb = best >= _POS_T
        negb = best < _NEG_T
        posf = posb.astype(f32)

        po = pred_ref[0, a * _KD + 4]
        obj_loss = (jnp.maximum(po, 0.0) - po * posf
                    + jnp.log1p(jnp.exp(-jnp.abs(po))))
        num_pos += jnp.sum(posf)
        num_neg += jnp.sum(negb.astype(f32))
        obj_pos_sum += jnp.sum(obj_loss * posf)
        neg_bits.append(lax.bitcast_convert_type(
            jnp.where(negb, obj_loss, -1.0), jnp.int32))
        obj_losses.append(obj_loss)

        pc0 = pred_ref[0, a * _KD + 5]
        pc1 = pred_ref[0, a * _KD + 6]
        pc2 = pred_ref[0, a * _KD + 7]
        m = jnp.maximum(jnp.maximum(pc0, pc1), pc2)
        lse = m + jnp.log(jnp.exp(pc0 - m) + jnp.exp(pc1 - m)
                          + jnp.exp(pc2 - m))

        pb0 = pred_ref[0, a * _KD + 0]
        pb1 = pred_ref[0, a * _KD + 1]
        pb2 = pred_ref[0, a * _KD + 2]
        pb3 = pred_ref[0, a * _KD + 3]

        def cl_body(g, carry, bg=bg, posf=posf, pc0=pc0, pc1=pc1, pc2=pc2,
                    lse=lse, axc=axc, ayc=ayc, aw=aw, ah=ah,
                    pb0=pb0, pb1=pb1, pb2=pb2, pb3=pb3):
            cls_acc, loc_acc = carry
            maskf = (bg == g.astype(f32)).astype(f32) * posf
            lab = lab_ref[0, g]
            t = jnp.clip(lab - 1, 0, _NUM_CLASSES - 1)
            sel = jnp.where(t == 0, pc0, jnp.where(t == 1, pc1, pc2))
            cls_acc = cls_acc + (lse - sel) * maskf
            base = 4 * g
            bx1 = gtb_ref[0, base]
            by1 = gtb_ref[0, base + 1]
            bx2 = gtb_ref[0, base + 2]
            by2 = gtb_ref[0, base + 3]
            gx = (bx1 + bx2) * 0.5
            gy = (by1 + by2) * 0.5
            gw = jnp.maximum(bx2 - bx1, 1e-6)
            gh = jnp.maximum(by2 - by1, 1e-6)
            tx = (gx - axc) / aw
            ty = (gy - ayc) / ah
            tw = jnp.log(gw / aw)
            th = jnp.log(gh / ah)
            sl = (_smooth_l1(pb0, tx) + _smooth_l1(pb1, ty)
                  + _smooth_l1(pb2, tw) + _smooth_l1(pb3, th))
            loc_acc = loc_acc + sl * maskf
            return cls_acc, loc_acc

        zero = jnp.zeros((_HW, _HW), f32)
        cls_acc, loc_acc = lax.fori_loop(0, _G, cl_body, (zero, zero))
        cls_sum += jnp.sum(cls_acc)
        loc_sum += jnp.sum(loc_acc)

    np_i = num_pos.astype(jnp.int32)
    nn_i = num_neg.astype(jnp.int32)
    k_nopos = jnp.where(nn_i > 0, jnp.maximum(nn_i // 10, 1), 0)
    k_i = jnp.where(np_i == 0, k_nopos,
                    jnp.minimum(_RATIO * np_i, nn_i))
    k_f = k_i.astype(f32)

    def bs_body(_, lohi):
        lo, hi = lohi
        mid = lo + (hi - lo) // 2
        cnt = f32(0.0)
        for bits in neg_bits:
            cnt += jnp.sum((bits >= mid).astype(f32))
        ge = cnt >= k_f
        return jnp.where(ge, mid, lo), jnp.where(ge, hi, mid)

    lo, _hi = lax.fori_loop(
        0, 31, bs_body, (jnp.int32(0), jnp.int32(0x7F800000)))

    cnt_gt = f32(0.0)
    sum_gt = f32(0.0)
    vk = f32(-1.0)
    for bits, ol in zip(neg_bits, obj_losses):
        gt = bits > lo
        cnt_gt += jnp.sum(gt.astype(f32))
        sum_gt += jnp.sum(jnp.where(gt, ol, 0.0))
        vk = jnp.maximum(vk, jnp.max(jnp.where(bits == lo, ol, -1.0)))
    topk_sum = jnp.where(k_i > 0, sum_gt + (k_f - cnt_gt) * vk, 0.0)

    obj_total = obj_pos_sum + topk_sum
    nsel = num_pos + k_f

    lane = lax.broadcasted_iota(jnp.int32, (1, _HW), 1)
    vec = (jnp.where(lane == 0, obj_total, 0.0)
           + jnp.where(lane == 1, cls_sum, 0.0)
           + jnp.where(lane == 2, loc_sum, 0.0)
           + jnp.where(lane == 3, num_pos, 0.0)
           + jnp.where(lane == 4, nsel, 0.0))

    @pl.when(b == 0)
    def _():
        out_ref[...] = vec

    @pl.when(b > 0)
    def _():
        out_ref[...] += vec


@jax.jit
def kernel(pred, anchors, gt_boxes, gt_labels):
    B = pred.shape[0]
    anc = anchors.reshape(_HW, _HW, _A, 4).transpose(2, 3, 0, 1)
    gtb = gt_boxes.reshape(B, _G * 4)
    lab = gt_labels.astype(jnp.int32)

    out = pl.pallas_call(
        _loss_kernel,
        grid=(B,),
        in_specs=[
            pl.BlockSpec((1, _A * _KD, _HW, _HW), lambda b: (b, 0, 0, 0)),
            pl.BlockSpec((_A, 4, _HW, _HW), lambda b: (0, 0, 0, 0)),
            pl.BlockSpec((1, _G * 4), lambda b: (b, 0),
                         memory_space=pltpu.SMEM),
            pl.BlockSpec((1, _G), lambda b: (b, 0),
                         memory_space=pltpu.SMEM),
        ],
        out_specs=pl.BlockSpec((1, _HW), lambda b: (0, 0)),
        out_shape=jax.ShapeDtypeStruct((1, _HW), jnp.float32),
    )(pred, anc, gtb, lab)

    r = out[0]
    obj_s, cls_s, loc_s, tp, ts = r[0], r[1], r[2], r[3], r[4]
    denom_pos = jnp.maximum(tp, 1.0)
    denom_obj = jnp.maximum(ts, 1.0)
    loss_loc = loc_s / denom_pos
    loss_cls = cls_s / denom_pos
    loss_obj = obj_s / denom_obj
    loss_total = 2.0 * loss_loc + 1.0 * loss_cls + 1.0 * loss_obj
    return (loss_obj, loss_cls, loss_loc, loss_total)


# speedup vs baseline: 42.5409x; 42.5409x over previous
import jax
import jax.numpy as jnp
from jax import lax
from jax.experimental import pallas as pl
from jax.experimental.pallas import tpu as pltpu

_NUM_CLASSES = 3
_POS_T, _NEG_T = 0.5, 0.3
_RATIO = 3
_A = 3
_G = 32
_HW = 128
_KD = 5 + _NUM_CLASSES


def _smooth_l1(x, y):
    d = jnp.abs(x - y)
    return jnp.where(d < 1.0, 0.5 * d * d, d - 0.5)


def _loss_kernel(pred_ref, anc_ref, gtb_ref, lab_ref, out_ref):
    b = pl.program_id(0)
    f32 = jnp.float32

    num_pos = f32(0.0)
    num_neg = f32(0.0)
    obj_pos_sum = f32(0.0)
    cls_sum = f32(0.0)
    loc_sum = f32(0.0)

    neg_bits = []
    obj_losses = []

    for a in range(_A):
        ax1 = anc_ref[a, 0]
        ay1 = anc_ref[a, 1]
        ax2 = anc_ref[a, 2]
        ay2 = anc_ref[a, 3]
        area_a = jnp.maximum(ax2 - ax1, 0.0) * jnp.maximum(ay2 - ay1, 0.0)
        aw = jnp.maximum(ax2 - ax1, 1e-6)
        ah = jnp.maximum(ay2 - ay1, 1e-6)
        axc = (ax1 + ax2) * 0.5
        ayc = (ay1 + ay2) * 0.5

        def iou_body(g, carry, ax1=ax1, ay1=ay1, ax2=ax2, ay2=ay2,
                     area_a=area_a):
            best, bg = carry
            base = 4 * g
            bx1 = gtb_ref[0, 0, base]
            by1 = gtb_ref[0, 0, base + 1]
            bx2 = gtb_ref[0, 0, base + 2]
            by2 = gtb_ref[0, 0, base + 3]
            ix1 = jnp.maximum(ax1, bx1)
            iy1 = jnp.maximum(ay1, by1)
            ix2 = jnp.minimum(ax2, bx2)
            iy2 = jnp.minimum(ay2, by2)
            inter = jnp.maximum(ix2 - ix1, 0.0) * jnp.maximum(iy2 - iy1, 0.0)
            ab = jnp.maximum(bx2 - bx1, 0.0) * jnp.maximum(by2 - by1, 0.0)
            union = area_a + ab - inter
            iou = inter / jnp.maximum(union, 1e-9)
            upd = iou > best
            best = jnp.where(upd, iou, best)
            bg = jnp.where(upd, g.astype(f32), bg)
            return best, bg

        init = (jnp.full((_HW, _HW), -1.0, f32), jnp.zeros((_HW, _HW), f32))
        best, bg = lax.fori_loop(0, _G, iou_body, init)

        posb = best >= _POS_T
        negb = best < _NEG_T
        posf = posb.astype(f32)

        po = pred_ref[0, a * _KD + 4]
        obj_loss = (jnp.maximum(po, 0.0) - po * posf
                    + jnp.log1p(jnp.exp(-jnp.abs(po))))
        num_pos += jnp.sum(posf)
        num_neg += jnp.sum(negb.astype(f32))
        obj_pos_sum += jnp.sum(obj_loss * posf)
        neg_bits.append(lax.bitcast_convert_type(
            jnp.where(negb, obj_loss, -1.0), jnp.int32))
        obj_losses.append(obj_loss)

        pc0 = pred_ref[0, a * _KD + 5]
        pc1 = pred_ref[0, a * _KD + 6]
        pc2 = pred_ref[0, a * _KD + 7]
        m = jnp.maximum(jnp.maximum(pc0, pc1), pc2)
        lse = m + jnp.log(jnp.exp(pc0 - m) + jnp.exp(pc1 - m)
                          + jnp.exp(pc2 - m))

        pb0 = pred_ref[0, a * _KD + 0]
        pb1 = pred_ref[0, a * _KD + 1]
        pb2 = pred_ref[0, a * _KD + 2]
        pb3 = pred_ref[0, a * _KD + 3]

        def cl_body(g, carry, bg=bg, posf=posf, pc0=pc0, pc1=pc1, pc2=pc2,
                    lse=lse, axc=axc, ayc=ayc, aw=aw, ah=ah,
                    pb0=pb0, pb1=pb1, pb2=pb2, pb3=pb3):
            cls_acc, loc_acc = carry
            maskf = (bg == g.astype(f32)).astype(f32) * posf
            lab = lab_ref[0, 0, g]
            t = jnp.clip(lab - 1, 0, _NUM_CLASSES - 1)
            sel = jnp.where(t == 0, pc0, jnp.where(t == 1, pc1, pc2))
            cls_acc = cls_acc + (lse - sel) * maskf
            base = 4 * g
            bx1 = gtb_ref[0, 0, base]
            by1 = gtb_ref[0, 0, base + 1]
            bx2 = gtb_ref[0, 0, base + 2]
            by2 = gtb_ref[0, 0, base + 3]
            gx = (bx1 + bx2) * 0.5
            gy = (by1 + by2) * 0.5
            gw = jnp.maximum(bx2 - bx1, 1e-6)
            gh = jnp.maximum(by2 - by1, 1e-6)
            tx = (gx - axc) / aw
            ty = (gy - ayc) / ah
            tw = jnp.log(gw / aw)
            th = jnp.log(gh / ah)
            sl = (_smooth_l1(pb0, tx) + _smooth_l1(pb1, ty)
                  + _smooth_l1(pb2, tw) + _smooth_l1(pb3, th))
            loc_acc = loc_acc + sl * maskf
            return cls_acc, loc_acc

        zero = jnp.zeros((_HW, _HW), f32)
        cls_acc, loc_acc = lax.fori_loop(0, _G, cl_body, (zero, zero))
        cls_sum += jnp.sum(cls_acc)
        loc_sum += jnp.sum(loc_acc)

    np_i = num_pos.astype(jnp.int32)
    nn_i = num_neg.astype(jnp.int32)
    k_nopos = jnp.where(nn_i > 0, jnp.maximum(nn_i // 10, 1), 0)
    k_i = jnp.where(np_i == 0, k_nopos,
                    jnp.minimum(_RATIO * np_i, nn_i))
    k_f = k_i.astype(f32)

    def bs_body(_, lohi):
        lo, hi = lohi
        mid = lo + (hi - lo) // 2
        cnt = f32(0.0)
        for bits in neg_bits:
            cnt += jnp.sum((bits >= mid).astype(f32))
        ge = cnt >= k_f
        return jnp.where(ge, mid, lo), jnp.where(ge, hi, mid)

    lo, _hi = lax.fori_loop(
        0, 31, bs_body, (jnp.int32(0), jnp.int32(0x7F800000)))

    cnt_gt = f32(0.0)
    sum_gt = f32(0.0)
    vk = f32(-1.0)
    for bits, ol in zip(neg_bits, obj_losses):
        gt = bits > lo
        cnt_gt += jnp.sum(gt.astype(f32))
        sum_gt += jnp.sum(jnp.where(gt, ol, 0.0))
        vk = jnp.maximum(vk, jnp.max(jnp.where(bits == lo, ol, -1.0)))
    topk_sum = jnp.where(k_i > 0, sum_gt + (k_f - cnt_gt) * vk, 0.0)

    obj_total = obj_pos_sum + topk_sum
    nsel = num_pos + k_f

    lane = lax.broadcasted_iota(jnp.int32, (1, _HW), 1)
    vec = (jnp.where(lane == 0, obj_total, 0.0)
           + jnp.where(lane == 1, cls_sum, 0.0)
           + jnp.where(lane == 2, loc_sum, 0.0)
           + jnp.where(lane == 3, num_pos, 0.0)
           + jnp.where(lane == 4, nsel, 0.0))

    @pl.when(b == 0)
    def _():
        out_ref[...] = vec

    @pl.when(b > 0)
    def _():
        out_ref[...] += vec


@jax.jit
def kernel(pred, anchors, gt_boxes, gt_labels):
    B = pred.shape[0]
    anc = anchors.reshape(_HW, _HW, _A, 4).transpose(2, 3, 0, 1)
    gtb = gt_boxes.reshape(B, 1, _G * 4)
    lab = gt_labels.astype(jnp.int32).reshape(B, 1, _G)

    out = pl.pallas_call(
        _loss_kernel,
        grid=(B,),
        in_specs=[
            pl.BlockSpec((1, _A * _KD, _HW, _HW), lambda b: (b, 0, 0, 0)),
            pl.BlockSpec((_A, 4, _HW, _HW), lambda b: (0, 0, 0, 0)),
            pl.BlockSpec((1, 1, _G * 4), lambda b: (b, 0, 0),
                         memory_space=pltpu.SMEM),
            pl.BlockSpec((1, 1, _G), lambda b: (b, 0, 0),
                         memory_space=pltpu.SMEM),
        ],
        out_specs=pl.BlockSpec((1, _HW), lambda b: (0, 0)),
        out_shape=jax.ShapeDtypeStruct((1, _HW), jnp.float32),
    )(pred, anc, gtb, lab)

    r = out[0]
    obj_s, cls_s, loc_s, tp, ts = r[0], r[1], r[2], r[3], r[4]
    denom_pos = jnp.maximum(tp, 1.0)
    denom_obj = jnp.maximum(ts, 1.0)
    loss_loc = loc_s / denom_pos
    loss_cls = cls_s / denom_pos
    loss_obj = obj_s / denom_obj
    loss_total = 2.0 * loss_loc + 1.0 * loss_cls + 1.0 * loss_obj
    return (loss_obj, loss_cls, loss_loc, loss_total)


# speedup vs baseline: 57.2153x; 1.3449x over previous
import jax
import jax.numpy as jnp
from jax import lax
from jax.experimental import pallas as pl
from jax.experimental.pallas import tpu as pltpu

_NUM_CLASSES = 3
_POS_T, _NEG_T = 0.5, 0.3
_RATIO = 3
_A = 3
_G = 32
_HW = 128
_KD = 5 + _NUM_CLASSES


def _smooth_l1(x, y):
    d = jnp.abs(x - y)
    return jnp.where(d < 1.0, 0.5 * d * d, d - 0.5)


def _loss_kernel(pred_ref, anc_ref, gtb_ref, lab_ref, out_ref):
    b = pl.program_id(0)
    f32 = jnp.float32

    num_pos = f32(0.0)
    num_neg = f32(0.0)
    obj_pos_sum = f32(0.0)
    cls_sum = f32(0.0)
    loc_sum = f32(0.0)

    neg_bits = []
    obj_losses = []

    for a in range(_A):
        ax1 = anc_ref[a, 0]
        ay1 = anc_ref[a, 1]
        ax2 = anc_ref[a, 2]
        ay2 = anc_ref[a, 3]
        area_a = jnp.maximum(ax2 - ax1, 0.0) * jnp.maximum(ay2 - ay1, 0.0)
        aw = jnp.maximum(ax2 - ax1, 1e-6)
        ah = jnp.maximum(ay2 - ay1, 1e-6)
        axc = (ax1 + ax2) * 0.5
        ayc = (ay1 + ay2) * 0.5

        def iou_body(g, carry, ax1=ax1, ay1=ay1, ax2=ax2, ay2=ay2,
                     area_a=area_a):
            best, bg = carry
            base = 4 * g
            bx1 = gtb_ref[0, 0, base]
            by1 = gtb_ref[0, 0, base + 1]
            bx2 = gtb_ref[0, 0, base + 2]
            by2 = gtb_ref[0, 0, base + 3]
            ix1 = jnp.maximum(ax1, bx1)
            iy1 = jnp.maximum(ay1, by1)
            ix2 = jnp.minimum(ax2, bx2)
            iy2 = jnp.minimum(ay2, by2)
            inter = jnp.maximum(ix2 - ix1, 0.0) * jnp.maximum(iy2 - iy1, 0.0)
            ab = jnp.maximum(bx2 - bx1, 0.0) * jnp.maximum(by2 - by1, 0.0)
            union = area_a + ab - inter
            iou = inter / jnp.maximum(union, 1e-9)
            upd = iou > best
            best = jnp.where(upd, iou, best)
            bg = jnp.where(upd, g.astype(f32), bg)
            return best, bg

        init = (jnp.full((_HW, _HW), -1.0, f32), jnp.zeros((_HW, _HW), f32))
        best, bg = lax.fori_loop(0, _G, iou_body, init)

        posb = best >= _POS_T
        negb = best < _NEG_T
        posf = posb.astype(f32)

        po = pred_ref[0, a * _KD + 4]
        obj_loss = (jnp.maximum(po, 0.0) - po * posf
                    + jnp.log1p(jnp.exp(-jnp.abs(po))))
        num_pos += jnp.sum(posf)
        num_neg += jnp.sum(negb.astype(f32))
        obj_pos_sum += jnp.sum(obj_loss * posf)
        neg_bits.append(lax.bitcast_convert_type(
            jnp.where(negb, obj_loss, -1.0), jnp.int32))
        obj_losses.append(obj_loss)

        pc0 = pred_ref[0, a * _KD + 5]
        pc1 = pred_ref[0, a * _KD + 6]
        pc2 = pred_ref[0, a * _KD + 7]
        m = jnp.maximum(jnp.maximum(pc0, pc1), pc2)
        lse = m + jnp.log(jnp.exp(pc0 - m) + jnp.exp(pc1 - m)
                          + jnp.exp(pc2 - m))

        pb0 = pred_ref[0, a * _KD + 0]
        pb1 = pred_ref[0, a * _KD + 1]
        pb2 = pred_ref[0, a * _KD + 2]
        pb3 = pred_ref[0, a * _KD + 3]

        def gather_body(g, carry, bg=bg):
            gxm, gym, gwm, ghm, tm = carry
            m = bg == g.astype(f32)
            base = 4 * g
            bx1 = gtb_ref[0, 0, base]
            by1 = gtb_ref[0, 0, base + 1]
            bx2 = gtb_ref[0, 0, base + 2]
            by2 = gtb_ref[0, 0, base + 3]
            lab = lab_ref[0, 0, g]
            t = jnp.clip(lab - 1, 0, _NUM_CLASSES - 1).astype(f32)
            gxm = jnp.where(m, (bx1 + bx2) * 0.5, gxm)
            gym = jnp.where(m, (by1 + by2) * 0.5, gym)
            gwm = jnp.where(m, jnp.maximum(bx2 - bx1, 1e-6), gwm)
            ghm = jnp.where(m, jnp.maximum(by2 - by1, 1e-6), ghm)
            tm = jnp.where(m, t, tm)
            return gxm, gym, gwm, ghm, tm

        zero = jnp.zeros((_HW, _HW), f32)
        gxm, gym, gwm, ghm, tm = lax.fori_loop(
            0, _G, gather_body, (zero, zero, zero, zero, zero))

        sel = jnp.where(tm == 0.0, pc0, jnp.where(tm == 1.0, pc1, pc2))
        cls_sum += jnp.sum((lse - sel) * posf)

        tx = (gxm - axc) / aw
        ty = (gym - ayc) / ah
        tw = jnp.log(gwm / aw)
        th = jnp.log(ghm / ah)
        sl = (_smooth_l1(pb0, tx) + _smooth_l1(pb1, ty)
              + _smooth_l1(pb2, tw) + _smooth_l1(pb3, th))
        loc_sum += jnp.sum(sl * posf)

    np_i = num_pos.astype(jnp.int32)
    nn_i = num_neg.astype(jnp.int32)
    k_nopos = jnp.where(nn_i > 0, jnp.maximum(nn_i // 10, 1), 0)
    k_i = jnp.where(np_i == 0, k_nopos,
                    jnp.minimum(_RATIO * np_i, nn_i))
    k_f = k_i.astype(f32)

    def bs_body(_, lohi):
        lo, hi = lohi
        mid = lo + (hi - lo) // 2
        cnt = f32(0.0)
        for bits in neg_bits:
            cnt += jnp.sum((bits >= mid).astype(f32))
        ge = cnt >= k_f
        return jnp.where(ge, mid, lo), jnp.where(ge, hi, mid)

    lo, _hi = lax.fori_loop(
        0, 31, bs_body, (jnp.int32(0), jnp.int32(0x7F800000)))

    cnt_gt = f32(0.0)
    sum_gt = f32(0.0)
    vk = f32(-1.0)
    for bits, ol in zip(neg_bits, obj_losses):
        gt = bits > lo
        cnt_gt += jnp.sum(gt.astype(f32))
        sum_gt += jnp.sum(jnp.where(gt, ol, 0.0))
        vk = jnp.maximum(vk, jnp.max(jnp.where(bits == lo, ol, -1.0)))
    topk_sum = jnp.where(k_i > 0, sum_gt + (k_f - cnt_gt) * vk, 0.0)

    obj_total = obj_pos_sum + topk_sum
    nsel = num_pos + k_f

    lane = lax.broadcasted_iota(jnp.int32, (1, _HW), 1)
    vec = (jnp.where(lane == 0, obj_total, 0.0)
           + jnp.where(lane == 1, cls_sum, 0.0)
           + jnp.where(lane == 2, loc_sum, 0.0)
           + jnp.where(lane == 3, num_pos, 0.0)
           + jnp.where(lane == 4, nsel, 0.0))

    @pl.when(b == 0)
    def _():
        out_ref[...] = vec

    @pl.when(b > 0)
    def _():
        out_ref[...] += vec


@jax.jit
def kernel(pred, anchors, gt_boxes, gt_labels):
    B = pred.shape[0]
    anc = anchors.reshape(_HW, _HW, _A, 4).transpose(2, 3, 0, 1)
    gtb = gt_boxes.reshape(B, 1, _G * 4)
    lab = gt_labels.astype(jnp.int32).reshape(B, 1, _G)

    out = pl.pallas_call(
        _loss_kernel,
        grid=(B,),
        in_specs=[
            pl.BlockSpec((1, _A * _KD, _HW, _HW), lambda b: (b, 0, 0, 0)),
            pl.BlockSpec((_A, 4, _HW, _HW), lambda b: (0, 0, 0, 0)),
            pl.BlockSpec((1, 1, _G * 4), lambda b: (b, 0, 0),
                         memory_space=pltpu.SMEM),
            pl.BlockSpec((1, 1, _G), lambda b: (b, 0, 0),
                         memory_space=pltpu.SMEM),
        ],
        out_specs=pl.BlockSpec((1, _HW), lambda b: (0, 0)),
        out_shape=jax.ShapeDtypeStruct((1, _HW), jnp.float32),
    )(pred, anc, gtb, lab)

    r = out[0]
    obj_s, cls_s, loc_s, tp, ts = r[0], r[1], r[2], r[3], r[4]
    denom_pos = jnp.maximum(tp, 1.0)
    denom_obj = jnp.maximum(ts, 1.0)
    loss_loc = loc_s / denom_pos
    loss_cls = cls_s / denom_pos
    loss_obj = obj_s / denom_obj
    loss_total = 2.0 * loss_loc + 1.0 * loss_cls + 1.0 * loss_obj
    return (loss_obj, loss_cls, loss_loc, loss_total)


# speedup vs baseline: 60.6830x; 1.0606x over previous
import jax
import jax.numpy as jnp
from jax import lax
from jax.experimental import pallas as pl
from jax.experimental.pallas import tpu as pltpu

_NUM_CLASSES = 3
_POS_T, _NEG_T = 0.5, 0.3
_RATIO = 3
_A = 3
_G = 32
_HW = 128
_KD = 5 + _NUM_CLASSES


def _smooth_l1(x, y):
    d = jnp.abs(x - y)
    return jnp.where(d < 1.0, 0.5 * d * d, d - 0.5)


def _loss_kernel(pred_ref, anc_ref, gtb_ref, lab_ref, out_ref):
    b = pl.program_id(0)
    f32 = jnp.float32

    ax1 = [anc_ref[a, 0] for a in range(_A)]
    ay1 = [anc_ref[a, 1] for a in range(_A)]
    ax2 = [anc_ref[a, 2] for a in range(_A)]
    ay2 = [anc_ref[a, 3] for a in range(_A)]
    area_a = [jnp.maximum(ax2[a] - ax1[a], 0.0)
              * jnp.maximum(ay2[a] - ay1[a], 0.0) for a in range(_A)]
    aw = [jnp.maximum(ax2[a] - ax1[a], 1e-6) for a in range(_A)]
    ah = [jnp.maximum(ay2[a] - ay1[a], 1e-6) for a in range(_A)]
    axc = [(ax1[a] + ax2[a]) * 0.5 for a in range(_A)]
    ayc = [(ay1[a] + ay2[a]) * 0.5 for a in range(_A)]

    def iou_body(g, carry):
        bests, bgs = carry
        base = 4 * g
        bx1 = gtb_ref[0, 0, base]
        by1 = gtb_ref[0, 0, base + 1]
        bx2 = gtb_ref[0, 0, base + 2]
        by2 = gtb_ref[0, 0, base + 3]
        ab = jnp.maximum(bx2 - bx1, 0.0) * jnp.maximum(by2 - by1, 0.0)
        gf = g.astype(f32)
        nb, ng = [], []
        for a in range(_A):
            ix1 = jnp.maximum(ax1[a], bx1)
            iy1 = jnp.maximum(ay1[a], by1)
            ix2 = jnp.minimum(ax2[a], bx2)
            iy2 = jnp.minimum(ay2[a], by2)
            inter = (jnp.maximum(ix2 - ix1, 0.0)
                     * jnp.maximum(iy2 - iy1, 0.0))
            iou = inter / (area_a[a] + ab - inter)
            upd = iou > bests[a]
            nb.append(jnp.where(upd, iou, bests[a]))
            ng.append(jnp.where(upd, gf, bgs[a]))
        return tuple(nb), tuple(ng)

    neg1 = jnp.full((_HW, _HW), -1.0, f32)
    zero = jnp.zeros((_HW, _HW), f32)
    bests, bgs = lax.fori_loop(
        0, _G, iou_body, ((neg1,) * _A, (zero,) * _A))

    posf = [(bests[a] >= _POS_T).astype(f32) for a in range(_A)]
    negb = [bests[a] < _NEG_T for a in range(_A)]

    po = [pred_ref[0, a * _KD + 4] for a in range(_A)]
    obj_loss = [(jnp.maximum(po[a], 0.0) - po[a] * posf[a]
                 + jnp.log1p(jnp.exp(-jnp.abs(po[a])))) for a in range(_A)]
    neg_bits = [lax.bitcast_convert_type(
        jnp.where(negb[a], obj_loss[a], -1.0), jnp.int32) for a in range(_A)]

    pos_plane = posf[0] + posf[1] + posf[2]
    neg_plane = (negb[0].astype(f32) + negb[1].astype(f32)
                 + negb[2].astype(f32))
    objpos_plane = (obj_loss[0] * posf[0] + obj_loss[1] * posf[1]
                    + obj_loss[2] * posf[2])
    num_pos = jnp.sum(pos_plane)
    num_neg = jnp.sum(neg_plane)
    obj_pos_sum = jnp.sum(objpos_plane)

    np_i = num_pos.astype(jnp.int32)
    nn_i = num_neg.astype(jnp.int32)
    k_nopos = jnp.where(nn_i > 0, jnp.maximum(nn_i // 10, 1), 0)
    k_i = jnp.where(np_i == 0, k_nopos, jnp.minimum(_RATIO * np_i, nn_i))
    k_f = k_i.astype(f32)

    def body(g, carry):
        gxm, gym, gwm, ghm, tm, lo, hi = carry
        base = 4 * g
        bx1 = gtb_ref[0, 0, base]
        by1 = gtb_ref[0, 0, base + 1]
        bx2 = gtb_ref[0, 0, base + 2]
        by2 = gtb_ref[0, 0, base + 3]
        lab = lab_ref[0, 0, g]
        t = jnp.clip(lab - 1, 0, _NUM_CLASSES - 1).astype(f32)
        gx = (bx1 + bx2) * 0.5
        gy = (by1 + by2) * 0.5
        gw = jnp.maximum(bx2 - bx1, 1e-6)
        gh = jnp.maximum(by2 - by1, 1e-6)
        gf = g.astype(f32)
        gxm2, gym2, gwm2, ghm2, tm2 = [], [], [], [], []
        for a in range(_A):
            m = bgs[a] == gf
            gxm2.append(jnp.where(m, gx, gxm[a]))
            gym2.append(jnp.where(m, gy, gym[a]))
            gwm2.append(jnp.where(m, gw, gwm[a]))
            ghm2.append(jnp.where(m, gh, ghm[a]))
            tm2.append(jnp.where(m, t, tm[a]))
        mid = lo + (hi - lo) // 2
        cnt = ((neg_bits[0] >= mid).astype(f32)
               + (neg_bits[1] >= mid).astype(f32)
               + (neg_bits[2] >= mid).astype(f32))
        ge = jnp.sum(cnt) >= k_f
        lo2 = jnp.where(ge, mid, lo)
        hi2 = jnp.where(ge, hi, mid)
        return (tuple(gxm2), tuple(gym2), tuple(gwm2), tuple(ghm2),
                tuple(tm2), lo2, hi2)

    z3 = (zero,) * _A
    gxm, gym, gwm, ghm, tm, lo, _hi = lax.fori_loop(
        0, _G, body,
        (z3, z3, z3, z3, z3, jnp.int32(0), jnp.int32(0x7F800000)))

    cnt_gt_plane = zero
    sum_gt_plane = zero
    vk_plane = neg1
    for a in range(_A):
        gtm = neg_bits[a] > lo
        cnt_gt_plane += gtm.astype(f32)
        sum_gt_plane += jnp.where(gtm, obj_loss[a], 0.0)
        vk_plane = jnp.maximum(
            vk_plane, jnp.where(neg_bits[a] == lo, obj_loss[a], -1.0))
    cnt_gt = jnp.sum(cnt_gt_plane)
    sum_gt = jnp.sum(sum_gt_plane)
    vk = jnp.max(vk_plane)
    topk_sum = jnp.where(k_i > 0, sum_gt + (k_f - cnt_gt) * vk, 0.0)

    cls_plane = zero
    loc_plane = zero
    for a in range(_A):
        pc0 = pred_ref[0, a * _KD + 5]
        pc1 = pred_ref[0, a * _KD + 6]
        pc2 = pred_ref[0, a * _KD + 7]
        m = jnp.maximum(jnp.maximum(pc0, pc1), pc2)
        lse = m + jnp.log(jnp.exp(pc0 - m) + jnp.exp(pc1 - m)
                          + jnp.exp(pc2 - m))
        sel = jnp.where(tm[a] == 0.0, pc0,
                        jnp.where(tm[a] == 1.0, pc1, pc2))
        cls_plane += (lse - sel) * posf[a]

        tx = (gxm[a] - axc[a]) / aw[a]
        ty = (gym[a] - ayc[a]) / ah[a]
        tw = jnp.log(gwm[a] / aw[a])
        th = jnp.log(ghm[a] / ah[a])
        sl = (_smooth_l1(pred_ref[0, a * _KD + 0], tx)
              + _smooth_l1(pred_ref[0, a * _KD + 1], ty)
              + _smooth_l1(pred_ref[0, a * _KD + 2], tw)
              + _smooth_l1(pred_ref[0, a * _KD + 3], th))
        loc_plane += sl * posf[a]
    cls_sum = jnp.sum(cls_plane)
    loc_sum = jnp.sum(loc_plane)

    obj_total = obj_pos_sum + topk_sum
    nsel = num_pos + k_f

    lane = lax.broadcasted_iota(jnp.int32, (1, _HW), 1)
    vec = (jnp.where(lane == 0, obj_total, 0.0)
           + jnp.where(lane == 1, cls_sum, 0.0)
           + jnp.where(lane == 2, loc_sum, 0.0)
           + jnp.where(lane == 3, num_pos, 0.0)
           + jnp.where(lane == 4, nsel, 0.0))

    @pl.when(b == 0)
    def _():
        out_ref[...] = vec

    @pl.when(b > 0)
    def _():
        out_ref[...] += vec


@jax.jit
def kernel(pred, anchors, gt_boxes, gt_labels):
    B = pred.shape[0]
    anc = anchors.reshape(_HW, _HW, _A, 4).transpose(2, 3, 0, 1)
    gtb = gt_boxes.reshape(B, 1, _G * 4)
    lab = gt_labels.astype(jnp.int32).reshape(B, 1, _G)

    out = pl.pallas_call(
        _loss_kernel,
        grid=(B,),
        in_specs=[
            pl.BlockSpec((1, _A * _KD, _HW, _HW), lambda b: (b, 0, 0, 0)),
            pl.BlockSpec((_A, 4, _HW, _HW), lambda b: (0, 0, 0, 0)),
            pl.BlockSpec((1, 1, _G * 4), lambda b: (b, 0, 0),
                         memory_space=pltpu.SMEM),
            pl.BlockSpec((1, 1, _G), lambda b: (b, 0, 0),
                         memory_space=pltpu.SMEM),
        ],
        out_specs=pl.BlockSpec((1, _HW), lambda b: (0, 0)),
        out_shape=jax.ShapeDtypeStruct((1, _HW), jnp.float32),
    )(pred, anc, gtb, lab)

    r = out[0]
    obj_s, cls_s, loc_s, tp, ts = r[0], r[1], r[2], r[3], r[4]
    denom_pos = jnp.maximum(tp, 1.0)
    denom_obj = jnp.maximum(ts, 1.0)
    loss_loc = loc_s / denom_pos
    loss_cls = cls_s / denom_pos
    loss_obj = obj_s / denom_obj
    loss_total = 2.0 * loss_loc + 1.0 * loss_cls + 1.0 * loss_obj
    return (loss_obj, loss_cls, loss_loc, loss_total)
